# baseline (speedup 1.0000x reference)
import functools

import jax
import jax.numpy as jnp
import numpy as np
from jax import lax
from jax.experimental import pallas as pl
from jax.experimental.pallas import tpu as pltpu
from jax.experimental.pallas import tpu_sc as plsc

_B = 8
_C = 192
_H = 224
_W = 224
_ROWS = _B * _C
_K = 3

_RB = 48


def _sum_body(x_ref, o_ref):
    s2 = jnp.sum(x_ref[...], axis=2)
    o_ref[...] = jnp.sum(s2, axis=1, keepdims=True)


def _row_sums(x3d):
    return pl.pallas_call(
        _sum_body,
        grid=(_ROWS // _RB,),
        in_specs=[pl.BlockSpec((_RB, _H, _W), lambda r: (r, 0, 0))],
        out_specs=pl.BlockSpec((_RB, 1), lambda r: (r, 0)),
        out_shape=jax.ShapeDtypeStruct((_ROWS, 1), jnp.float32),
    )(x3d)


_NEG = np.float32(-3.0e38)
_BIG = np.int32(1 << 30)


def _sc_body(sums_hbm, w_hbm, x_hbm, out_hbm, ypad, yc, wv, rows, sem):
    cid = lax.axis_index("c")
    sid = lax.axis_index("s")
    w = sid * 2 + cid

    @pl.when(w < _B * _K)
    def _work():
        b = w // _K
        kk = w % _K

        iota = lax.iota(jnp.int32, 16)
        pltpu.sync_copy(w_hbm, wv)
        w0 = wv[pl.ds(0, 16)]
        w1 = wv[pl.ds(16, 16)]
        w2 = wv[pl.ds(32, 16)]

        zero = jnp.zeros((16,), jnp.float32)
        ypad[pl.ds(0, 16)] = zero
        ypad[pl.ds(192, 16)] = zero
        pltpu.sync_copy(sums_hbm.at[pl.ds(b * _C, _C)], ypad.at[pl.ds(8, _C)])

        for i in range(_C // 16):
            base = i * 16
            a = plsc.load_gather(ypad, [iota + (base + 7)])
            m = plsc.load_gather(ypad, [iota + (base + 8)])
            z = plsc.load_gather(ypad, [iota + (base + 9)])
            yc[pl.ds(base, 16)] = w0 * a + w1 * m + w2 * z

        sel = [np.int32(-1), np.int32(-1), np.int32(-1)]
        for k in range(_K):
            best_v = _NEG
            best_i = _BIG
            for i in range(_C // 16):
                v = yc[pl.ds(i * 16, 16)]
                g = iota + i * 16
                excl = (g == sel[0]) | (g == sel[1]) | (g == sel[2])
                v = jnp.where(excl, _NEG, v)
                m = jnp.max(v)
                ci = jnp.min(jnp.where(v == m, g, _BIG))
                take = m > best_v
                best_v = jnp.where(take, m, best_v)
                best_i = jnp.where(take, ci, best_i)
            sel[k] = best_i

        pick = jnp.where(kk == 0, sel[0], jnp.where(kk == 1, sel[1], sel[2]))
        pltpu.sync_copy(x_hbm.at[b, pick], out_hbm.at[b, kk])


@functools.cache
def _make_sc_call():
  return pl.kernel(
    _sc_body,
    out_type=jax.ShapeDtypeStruct((_B, _K, _H, _W), jnp.float32),
    mesh=plsc.VectorSubcoreMesh(core_axis_name="c", subcore_axis_name="s"),
    compiler_params=pltpu.CompilerParams(
        needs_layout_passes=False, use_tc_tiling_on_sc=True),
    scratch_types=[
        pltpu.VMEM((208,), jnp.float32),
        pltpu.VMEM((_C,), jnp.float32),
        pltpu.VMEM((48,), jnp.float32),
        pltpu.VMEM((_H, _W), jnp.float32),
        pltpu.SemaphoreType.DMA,
    ],
  )


def kernel(x, conv_w):
    x3d = x.reshape(_ROWS, _H, _W)
    sums = _row_sums(x3d).reshape(_ROWS)
    wpad = jnp.broadcast_to(conv_w.reshape(_K, 1), (_K, 16)).reshape(_K * 16)
    return _make_sc_call()(sums, wpad, x)


# speedup vs baseline: 2.2195x; 2.2195x over previous
import functools

import jax
import jax.numpy as jnp
import numpy as np
from jax import lax
from jax.experimental import pallas as pl
from jax.experimental.pallas import tpu as pltpu
from jax.experimental.pallas import tpu_sc as plsc

_B = 8
_C = 192
_H = 224
_W = 224
_ROWS = _B * _C
_K = 3

_RB = 48


def _sum_body(x_ref, o_ref):
    s2 = jnp.sum(x_ref[...], axis=2)
    o_ref[...] = jnp.sum(s2, axis=1, keepdims=True)


def _row_sums(x3d):
    return pl.pallas_call(
        _sum_body,
        grid=(_ROWS // _RB,),
        in_specs=[pl.BlockSpec((_RB, _H, _W), lambda r: (r, 0, 0))],
        out_specs=pl.BlockSpec((_RB, 1), lambda r: (r, 0)),
        out_shape=jax.ShapeDtypeStruct((_ROWS, 1), jnp.float32),
    )(x3d)


_NEG = np.float32(-3.0e38)
_BIG = np.int32(1 << 30)


def _sc_body(sums_hbm, w_hbm, x_hbm, out_hbm, ypad, yc, wv, rows, rows2, sem, sem2):
    cid = lax.axis_index("c")
    sid = lax.axis_index("s")
    w = sid * 2 + cid

    @pl.when(w < _B * _K)
    def _work():
        b = w // _K
        kk = w % _K

        iota = lax.iota(jnp.int32, 16)
        pltpu.sync_copy(w_hbm, wv)
        w0 = wv[pl.ds(0, 16)]
        w1 = wv[pl.ds(16, 16)]
        w2 = wv[pl.ds(32, 16)]

        zero = jnp.zeros((16,), jnp.float32)
        ypad[pl.ds(0, 16)] = zero
        ypad[pl.ds(192, 16)] = zero
        pltpu.sync_copy(sums_hbm.at[pl.ds(b * _C, _C)], ypad.at[pl.ds(8, _C)])

        for i in range(_C // 16):
            base = i * 16
            a = plsc.load_gather(ypad, [iota + (base + 7)])
            m = plsc.load_gather(ypad, [iota + (base + 8)])
            z = plsc.load_gather(ypad, [iota + (base + 9)])
            yc[pl.ds(base, 16)] = w0 * a + w1 * m + w2 * z

        sel = [np.int32(-1), np.int32(-1), np.int32(-1)]
        for k in range(_K):
            best_v = _NEG
            best_i = _BIG
            for i in range(_C // 16):
                v = yc[pl.ds(i * 16, 16)]
                g = iota + i * 16
                excl = (g == sel[0]) | (g == sel[1]) | (g == sel[2])
                v = jnp.where(excl, _NEG, v)
                m = jnp.max(v)
                ci = jnp.min(jnp.where(v == m, g, _BIG))
                take = m > best_v
                best_v = jnp.where(take, m, best_v)
                best_i = jnp.where(take, ci, best_i)
            sel[k] = best_i

        pick = jnp.where(kk == 0, sel[0], jnp.where(kk == 1, sel[1], sel[2]))
        g0 = pltpu.make_async_copy(
            x_hbm.at[b, pick, pl.ds(0, _H // 2)], rows, sem)
        g1 = pltpu.make_async_copy(
            x_hbm.at[b, pick, pl.ds(_H // 2, _H // 2)], rows2, sem2)
        g0.start()
        g1.start()
        g0.wait()
        p0 = pltpu.make_async_copy(
            rows, out_hbm.at[b, kk, pl.ds(0, _H // 2)], sem)
        p0.start()
        g1.wait()
        p1 = pltpu.make_async_copy(
            rows2, out_hbm.at[b, kk, pl.ds(_H // 2, _H // 2)], sem2)
        p1.start()
        p0.wait()
        p1.wait()


@functools.cache
def _make_sc_call():
  return pl.kernel(
    _sc_body,
    out_type=jax.ShapeDtypeStruct((_B, _K, _H, _W), jnp.float32),
    mesh=plsc.VectorSubcoreMesh(core_axis_name="c", subcore_axis_name="s"),
    compiler_params=pltpu.CompilerParams(
        needs_layout_passes=False, use_tc_tiling_on_sc=True),
    scratch_types=[
        pltpu.VMEM((208,), jnp.float32),
        pltpu.VMEM((_C,), jnp.float32),
        pltpu.VMEM((48,), jnp.float32),
        pltpu.VMEM((_H // 2, _W), jnp.float32),
        pltpu.VMEM((_H // 2, _W), jnp.float32),
        pltpu.SemaphoreType.DMA,
        pltpu.SemaphoreType.DMA,
    ],
  )


def kernel(x, conv_w):
    x3d = x.reshape(_ROWS, _H, _W)
    sums = _row_sums(x3d).reshape(_ROWS)
    wpad = jnp.broadcast_to(conv_w.reshape(_K, 1), (_K, 16)).reshape(_K * 16)
    return _make_sc_call()(sums, wpad, x)


# speedup vs baseline: 2.2362x; 1.0075x over previous
import functools

import jax
import jax.numpy as jnp
import numpy as np
from jax import lax
from jax.experimental import pallas as pl
from jax.experimental.pallas import tpu as pltpu
from jax.experimental.pallas import tpu_sc as plsc

_B = 8
_C = 192
_H = 224
_W = 224
_ROWS = _B * _C
_K = 3

_RB = 48


def _sum_body(x_ref, o_ref):
    s2 = jnp.sum(x_ref[0], axis=2)
    o_ref[...] = jnp.sum(s2, axis=1, keepdims=True)


def _row_sums(x):
    cb = _C // _RB
    return pl.pallas_call(
        _sum_body,
        grid=(_B, cb),
        in_specs=[pl.BlockSpec((1, _RB, _H, _W), lambda b, c: (b, c, 0, 0))],
        out_specs=pl.BlockSpec((_RB, 1), lambda b, c: (b * cb + c, 0)),
        out_shape=jax.ShapeDtypeStruct((_ROWS, 1), jnp.float32),
    )(x)


_NEG = np.float32(-3.0e38)
_BIG = np.int32(1 << 30)


def _sc_body(sums_hbm, w_hbm, x_hbm, out_hbm, ypad, yc, wv, rows, sem):
    cid = lax.axis_index("c")
    sid = lax.axis_index("s")
    w = sid * 2 + cid

    @pl.when(w < _B * _K)
    def _work():
        b = w // _K
        kk = w % _K

        iota = lax.iota(jnp.int32, 16)
        pltpu.sync_copy(w_hbm, wv)
        w0 = wv[pl.ds(0, 16)]
        w1 = wv[pl.ds(16, 16)]
        w2 = wv[pl.ds(32, 16)]

        zero = jnp.zeros((16,), jnp.float32)
        ypad[pl.ds(0, 16)] = zero
        ypad[pl.ds(192, 16)] = zero
        pltpu.sync_copy(sums_hbm.at[pl.ds(b * _C, _C)], ypad.at[pl.ds(8, _C)])

        for i in range(_C // 16):
            base = i * 16
            a = plsc.load_gather(ypad, [iota + (base + 7)])
            m = plsc.load_gather(ypad, [iota + (base + 8)])
            z = plsc.load_gather(ypad, [iota + (base + 9)])
            yc[pl.ds(base, 16)] = w0 * a + w1 * m + w2 * z

        sel = [np.int32(-1), np.int32(-1), np.int32(-1)]
        for k in range(_K):
            best_v = _NEG
            best_i = _BIG
            for i in range(_C // 16):
                v = yc[pl.ds(i * 16, 16)]
                g = iota + i * 16
                excl = (g == sel[0]) | (g == sel[1]) | (g == sel[2])
                v = jnp.where(excl, _NEG, v)
                m = jnp.max(v)
                ci = jnp.min(jnp.where(v == m, g, _BIG))
                take = m > best_v
                best_v = jnp.where(take, m, best_v)
                best_i = jnp.where(take, ci, best_i)
            sel[k] = best_i

        pick = jnp.where(kk == 0, sel[0], jnp.where(kk == 1, sel[1], sel[2]))
        pltpu.sync_copy(x_hbm.at[b, pick], rows)
        pltpu.sync_copy(rows, out_hbm.at[b, kk])


@functools.cache
def _make_sc_call():
  return pl.kernel(
    _sc_body,
    out_type=jax.ShapeDtypeStruct((_B, _K, _H, _W), jnp.float32),
    mesh=plsc.VectorSubcoreMesh(core_axis_name="c", subcore_axis_name="s"),
    compiler_params=pltpu.CompilerParams(
        needs_layout_passes=False, use_tc_tiling_on_sc=True),
    scratch_types=[
        pltpu.VMEM((208,), jnp.float32),
        pltpu.VMEM((_C,), jnp.float32),
        pltpu.VMEM((48,), jnp.float32),
        pltpu.VMEM((_H, _W), jnp.float32),
        pltpu.SemaphoreType.DMA,
    ],
  )


def kernel(x, conv_w):
    sums = _row_sums(x).reshape(_ROWS)
    wpad = jnp.broadcast_to(conv_w.reshape(_K, 1), (_K, 16)).reshape(_K * 16)
    return _make_sc_call()(sums, wpad, x)
